# baseline (speedup 1.0000x reference)
import jax
import jax.numpy as jnp
from jax import lax
from jax.experimental import pallas as pl
from jax.experimental.pallas import tpu as pltpu, tpu_sc as plsc

V = 1024
D = 768
B = 1024
S = 512

_info = plsc.get_sparse_core_info()
_NC, _NS = _info.num_cores, _info.num_subcores
_NW = _NC * _NS
_BPW = S // _NW

_TC_ROWS = B - S
_TC_BLK = 128


def _sc_gather_kernel(table_hbm, idx_hbm, out_hbm, idx_v, rows_v, sem):
    wid = lax.axis_index("s") * _NC + lax.axis_index("c")
    base = wid * _BPW
    pltpu.sync_copy(idx_hbm.at[pl.ds(base, _BPW)], idx_v)
    pltpu.async_copy(table_hbm.at[idx_v], rows_v, sem).wait()
    pltpu.sync_copy(rows_v, out_hbm.at[pl.ds(base, _BPW)])


def _tc_gather_body(idx_sref, idx_in_ref, table_ref, out_ref, idx_out_ref):
    out_ref[...] = table_ref[...]
    idx_out_ref[...] = idx_in_ref[...]


def kernel(table, position_ids):
    idx = position_ids.reshape(B).astype(jnp.int32)

    grid_spec = pltpu.PrefetchScalarGridSpec(
        num_scalar_prefetch=1,
        grid=(_TC_ROWS // _TC_BLK,),
        in_specs=[
            pl.BlockSpec((B,), lambda i, idx_sref: (0,)),
            pl.BlockSpec(
                (_TC_BLK, D),
                lambda i, idx_sref: (idx_sref[S + i * _TC_BLK] // _TC_BLK, 0),
            ),
        ],
        out_specs=[
            pl.BlockSpec((_TC_BLK, D), lambda i, idx_sref: (i, 0)),
            pl.BlockSpec((B,), lambda i, idx_sref: (0,)),
        ],
    )
    tc_part, idx_thru = pl.pallas_call(
        _tc_gather_body,
        grid_spec=grid_spec,
        out_shape=[
            jax.ShapeDtypeStruct((_TC_ROWS, D), table.dtype),
            jax.ShapeDtypeStruct((B,), jnp.int32),
        ],
    )(idx, idx, table)

    mesh = plsc.VectorSubcoreMesh(core_axis_name="c", subcore_axis_name="s")
    sc_gather = pl.kernel(
        _sc_gather_kernel,
        mesh=mesh,
        out_type=jax.ShapeDtypeStruct((S, D), table.dtype),
        scratch_types=[
            pltpu.VMEM((_BPW,), jnp.int32),
            pltpu.VMEM((_BPW, D), table.dtype),
            pltpu.SemaphoreType.DMA,
        ],
    )
    sc_part = sc_gather(table, idx_thru)

    out = jnp.concatenate([sc_part, tc_part], axis=0)
    return out.reshape(1, B, D)


# speedup vs baseline: 1.1973x; 1.1973x over previous
import jax
import jax.numpy as jnp
from jax import lax
from jax.experimental import pallas as pl
from jax.experimental.pallas import tpu as pltpu, tpu_sc as plsc

V = 1024
D = 768
B = 1024

_info = plsc.get_sparse_core_info()
_NC, _NS = _info.num_cores, _info.num_subcores
_NW = _NC * _NS
_BPW = B // _NW


def _gather_kernel(table_hbm, idx_hbm, out_hbm, idx_v, rows_v, sem):
    wid = lax.axis_index("s") * _NC + lax.axis_index("c")
    base = wid * _BPW
    pltpu.sync_copy(idx_hbm.at[pl.ds(base, _BPW)], idx_v)
    pltpu.async_copy(table_hbm.at[idx_v], rows_v, sem).wait()
    pltpu.sync_copy(rows_v, out_hbm.at[pl.ds(base, _BPW)])


def kernel(table, position_ids):
    idx = position_ids.reshape(B).astype(jnp.int32)
    mesh = plsc.VectorSubcoreMesh(core_axis_name="c", subcore_axis_name="s")
    gather = pl.kernel(
        _gather_kernel,
        mesh=mesh,
        out_type=jax.ShapeDtypeStruct((B, D), table.dtype),
        scratch_types=[
            pltpu.VMEM((_BPW,), jnp.int32),
            pltpu.VMEM((_BPW, D), table.dtype),
            pltpu.SemaphoreType.DMA,
        ],
    )
    out = gather(table, idx)
    return out.reshape(1, B, D)
